# baseline (speedup 1.0000x reference)
import functools

import jax
import jax.numpy as jnp
from jax import lax
from jax.experimental import pallas as pl
from jax.experimental.pallas import tpu as pltpu
from jax.experimental.pallas import tpu_sc as plsc

TOP_K = 2
TM = 768
DF_BLK = 128

SC_CORES = 2
SC_SUBCORES = 16
NW = SC_CORES * SC_SUBCORES
LANES = 16

GATHER_CHUNK = 16
GATHER_NBUF = 4


def _sc_gather_call(xb, row_token_3d, n_rows, hidden):
    rows_per_w = n_rows // NW
    chunk = GATHER_CHUNK
    n_chunks = rows_per_w // chunk
    idx_rows = rows_per_w // LANES
    nbuf = GATHER_NBUF

    mesh = plsc.VectorSubcoreMesh(
        core_axis_name="c", subcore_axis_name="s",
        num_cores=SC_CORES, num_subcores=SC_SUBCORES)

    @functools.partial(
        pl.kernel, mesh=mesh,
        out_type=jax.ShapeDtypeStruct((n_rows, hidden), jnp.int32),
        scratch_types=(
            [pltpu.VMEM((idx_rows, LANES), jnp.int32)]
            + [pltpu.VMEM((chunk, hidden), jnp.int32) for _ in range(nbuf)]
            + [pltpu.SemaphoreType.DMA for _ in range(2 * nbuf)]
        ),
    )
    def gk(x_hbm, idx_hbm, out_hbm, idx_v, *bufs_sems):
        bufs = bufs_sems[:nbuf]
        gsems = bufs_sems[nbuf:2 * nbuf]
        wsems = bufs_sems[2 * nbuf:]
        wid = lax.axis_index("s") * SC_CORES + lax.axis_index("c")
        base_row = wid * rows_per_w
        pltpu.sync_copy(idx_hbm.at[wid], idx_v)

        gdesc = [None] * nbuf
        wdesc = [None] * nbuf
        for c in range(min(nbuf, n_chunks)):
            gdesc[c] = pltpu.async_copy(
                x_hbm.at[idx_v.at[c]], bufs[c], gsems[c])
        for c in range(n_chunks):
            b = c % nbuf
            gdesc[b].wait()
            wdesc[b] = pltpu.async_copy(
                bufs[b], out_hbm.at[pl.ds(base_row + c * chunk, chunk)],
                wsems[b])
            nxt = c + nbuf
            if nxt < n_chunks:
                wdesc[b].wait()
                gdesc[b] = pltpu.async_copy(
                    x_hbm.at[idx_v.at[nxt]], bufs[b], gsems[b])
        for c in range(max(0, n_chunks - nbuf), n_chunks):
            b = c % nbuf
            if wdesc[b] is not None:
                wdesc[b].wait()
                wdesc[b] = None

    return gk(xb, row_token_3d)


def _sc_combine_call(ys, pos0_3d, pos1_3d, n_tok, hidden):
    tok_per_w = n_tok // NW
    chunk = 16
    n_chunks = tok_per_w // chunk
    idx_rows = tok_per_w // LANES

    mesh = plsc.VectorSubcoreMesh(
        core_axis_name="c", subcore_axis_name="s",
        num_cores=SC_CORES, num_subcores=SC_SUBCORES)

    @functools.partial(
        pl.kernel, mesh=mesh,
        out_type=jax.ShapeDtypeStruct((n_tok, hidden), jnp.float32),
        scratch_types=[
            pltpu.VMEM((idx_rows, LANES), jnp.int32),
            pltpu.VMEM((idx_rows, LANES), jnp.int32),
            pltpu.VMEM((chunk, hidden), jnp.float32),
            pltpu.VMEM((chunk, hidden), jnp.float32),
            pltpu.SemaphoreType.DMA,
            pltpu.SemaphoreType.DMA,
        ],
    )
    def ck(ys_hbm, p0_hbm, p1_hbm, out_hbm, p0_v, p1_v, buf0, buf1,
           sem0, sem1):
        wid = lax.axis_index("s") * SC_CORES + lax.axis_index("c")
        base_tok = wid * tok_per_w
        pltpu.sync_copy(p0_hbm.at[wid], p0_v)
        pltpu.sync_copy(p1_hbm.at[wid], p1_v)
        n_vec = hidden // LANES
        for c in range(n_chunks):
            d0 = pltpu.async_copy(ys_hbm.at[p0_v.at[c]], buf0, sem0)
            d1 = pltpu.async_copy(ys_hbm.at[p1_v.at[c]], buf1, sem1)
            d0.wait()
            d1.wait()

            def add_row(r, _):
                def add_vec(v, _):
                    sl = pl.ds(v * LANES, LANES)
                    buf0[r, sl] = buf0[r, sl] + buf1[r, sl]
                    return 0
                lax.fori_loop(0, n_vec, add_vec, 0)
                return 0
            lax.fori_loop(0, chunk, add_row, 0)
            pltpu.sync_copy(buf0,
                            out_hbm.at[pl.ds(base_tok + c * chunk, chunk)])

    return ck(ys, pos0_3d, pos1_3d)


def _ffn_kernel(tile_expert_ref, tile_flag_ref, tile_imap_ref,
                xs_ref, w1g_ref, w1u_ref, w2_ref, w_ref,
                ys_ref, acc_ref, *, n_j):
    j = pl.program_id(1)

    @pl.when(tile_flag_ref[pl.program_id(0)] != 0)
    def _active():
        xs = xs_ref[...]
        w1g = w1g_ref[0, 0].astype(jnp.bfloat16)
        w1u = w1u_ref[0, 0].astype(jnp.bfloat16)
        gate = lax.dot_general(
            xs, w1g, (((1,), (1,)), ((), ())),
            preferred_element_type=jnp.float32)
        up = lax.dot_general(
            xs, w1u, (((1,), (1,)), ((), ())),
            preferred_element_type=jnp.float32)
        act = (gate * jax.nn.sigmoid(gate) * up).astype(jnp.bfloat16)
        w2b = w2_ref[0].astype(jnp.bfloat16)
        yj = lax.dot_general(
            act, w2b, (((1,), (1,)), ((), ())),
            preferred_element_type=jnp.float32)

        @pl.when(j == 0)
        def _init():
            acc_ref[...] = yj

        @pl.when(j > 0)
        def _acc():
            acc_ref[...] += yj

        @pl.when(j == n_j - 1)
        def _weight():
            ys_ref[...] = acc_ref[...] * w_ref[...]


def kernel(x, router_logits, w1, w2):
    n_tok, hidden = x.shape
    n_exp = w1.shape[0]
    d_ff = w2.shape[2]

    probs = jax.nn.softmax(router_logits.astype(jnp.float32), axis=-1)
    topk_w, topk_idx = lax.top_k(probs, TOP_K)
    topk_w = topk_w / jnp.sum(topk_w, axis=-1, keepdims=True)

    n_asn = n_tok * TOP_K
    e_flat = topk_idx.reshape(-1).astype(jnp.int32)
    w_flat = topk_w.reshape(-1)
    t_flat = jnp.repeat(jnp.arange(n_tok, dtype=jnp.int32), TOP_K)

    order = jnp.argsort(e_flat)
    e_s = e_flat[order]
    t_s = t_flat[order]
    w_s = w_flat[order]

    counts = jnp.bincount(e_flat, length=n_exp)
    padded = ((counts + TM - 1) // TM) * TM
    pstart = jnp.cumsum(padded) - padded
    gstart = jnp.cumsum(counts) - counts
    rank = jnp.arange(n_asn, dtype=jnp.int32) - gstart[e_s].astype(jnp.int32)
    dest = pstart[e_s].astype(jnp.int32) + rank

    n_tiles = n_asn // TM + n_exp
    n_rows_tc = n_tiles * TM
    gq = NW * GATHER_CHUNK
    n_rows = ((n_rows_tc + gq - 1) // gq) * gq
    row_token = jnp.zeros((n_rows,), jnp.int32).at[dest].set(t_s)
    row_weight = jnp.zeros((n_rows_tc, 1), jnp.float32).at[dest, 0].set(w_s)

    flat_pos = jnp.zeros((n_asn,), jnp.int32).at[order].set(dest)
    pos = flat_pos.reshape(n_tok, TOP_K)
    pos0 = pos[:, 0].reshape(NW, n_tok // NW // LANES, LANES)
    pos1 = pos[:, 1].reshape(NW, n_tok // NW // LANES, LANES)

    tile_start = jnp.arange(n_tiles, dtype=jnp.int32) * TM
    total_padded = jnp.sum(padded).astype(jnp.int32)
    tile_flag = (tile_start < total_padded).astype(jnp.int32)
    n_valid = jnp.maximum(total_padded // TM, 1).astype(jnp.int32)
    pend = (pstart + padded).astype(jnp.int32)
    raw_expert = jnp.clip(
        jnp.searchsorted(pend, tile_start, side='right'), 0, n_exp - 1
    ).astype(jnp.int32)
    e_last = e_s[-1].astype(jnp.int32)
    tile_expert = jnp.where(tile_flag == 1, raw_expert, e_last)
    tile_imap = jnp.where(tile_flag == 1,
                          jnp.arange(n_tiles, dtype=jnp.int32), n_valid - 1)

    xb = x.astype(jnp.bfloat16)
    xbi = lax.bitcast_convert_type(
        xb.reshape(n_tok, hidden // 2, 2), jnp.int32)
    xs_i = _sc_gather_call(
        xbi, row_token.reshape(NW, n_rows // NW // LANES, LANES),
        n_rows, hidden // 2)
    xs = lax.bitcast_convert_type(xs_i, jnp.bfloat16).reshape(n_rows, hidden)

    w1r = w1.reshape(n_exp, 2, d_ff, hidden)
    n_j = d_ff // DF_BLK

    def wmap(sel):
        def f(i, j, te, tf, ti):
            je = jnp.where(tf[i] == 0, n_j - 1, j)
            return (te[i], sel, je, 0)
        return f

    def w2map(i, j, te, tf, ti):
        je = jnp.where(tf[i] == 0, n_j - 1, j)
        return (te[i], 0, je)

    grid_spec = pltpu.PrefetchScalarGridSpec(
        num_scalar_prefetch=3,
        grid=(n_tiles, n_j),
        in_specs=[
            pl.BlockSpec((TM, hidden), lambda i, j, te, tf, ti: (ti[i], 0)),
            pl.BlockSpec((1, 1, DF_BLK, hidden), wmap(0)),
            pl.BlockSpec((1, 1, DF_BLK, hidden), wmap(1)),
            pl.BlockSpec((1, hidden, DF_BLK), w2map),
            pl.BlockSpec((TM, 1), lambda i, j, te, tf, ti: (ti[i], 0)),
        ],
        out_specs=pl.BlockSpec((TM, hidden),
                               lambda i, j, te, tf, ti: (ti[i], 0)),
        scratch_shapes=[pltpu.VMEM((TM, hidden), jnp.float32)],
    )

    ys = pl.pallas_call(
        functools.partial(_ffn_kernel, n_j=n_j),
        grid_spec=grid_spec,
        out_shape=jax.ShapeDtypeStruct((n_rows_tc, hidden), jnp.float32),
        compiler_params=pltpu.CompilerParams(
            dimension_semantics=("arbitrary", "arbitrary")),
    )(tile_expert, tile_flag, tile_imap,
      xs, w1r, w1r, w2, row_weight)

    out = _sc_combine_call(ys, pos0, pos1, n_tok, hidden)
    return out


# speedup vs baseline: 1.0065x; 1.0065x over previous
import functools

import jax
import jax.numpy as jnp
from jax import lax
from jax.experimental import pallas as pl
from jax.experimental.pallas import tpu as pltpu
from jax.experimental.pallas import tpu_sc as plsc

TOP_K = 2
TM = 768
DF_BLK = 128

SC_CORES = 2
SC_SUBCORES = 16
NW = SC_CORES * SC_SUBCORES
LANES = 16

GATHER_CHUNK = 40
GATHER_NBUF = 2


def _sc_gather_call(xb, row_token_3d, n_rows, hidden):
    rows_per_w = n_rows // NW
    chunk = GATHER_CHUNK
    n_chunks = rows_per_w // chunk
    nbuf = GATHER_NBUF

    mesh = plsc.VectorSubcoreMesh(
        core_axis_name="c", subcore_axis_name="s",
        num_cores=SC_CORES, num_subcores=SC_SUBCORES)

    @functools.partial(
        pl.kernel, mesh=mesh,
        out_type=jax.ShapeDtypeStruct((n_rows, hidden), jnp.int32),
        scratch_types=(
            [pltpu.VMEM((n_chunks, chunk), jnp.int32)]
            + [pltpu.VMEM((chunk, hidden), jnp.int32) for _ in range(nbuf)]
            + [pltpu.SemaphoreType.DMA for _ in range(2 * nbuf)]
        ),
    )
    def gk(x_hbm, idx_hbm, out_hbm, idx_v, *bufs_sems):
        bufs = bufs_sems[:nbuf]
        gsems = bufs_sems[nbuf:2 * nbuf]
        wsems = bufs_sems[2 * nbuf:]
        wid = lax.axis_index("s") * SC_CORES + lax.axis_index("c")
        base_row = wid * rows_per_w
        pltpu.sync_copy(idx_hbm.at[wid], idx_v)

        gdesc = [None] * nbuf
        wdesc = [None] * nbuf
        for c in range(min(nbuf, n_chunks)):
            gdesc[c] = pltpu.async_copy(
                x_hbm.at[idx_v.at[c]], bufs[c], gsems[c])
        for c in range(n_chunks):
            b = c % nbuf
            gdesc[b].wait()
            wdesc[b] = pltpu.async_copy(
                bufs[b], out_hbm.at[pl.ds(base_row + c * chunk, chunk)],
                wsems[b])
            nxt = c + nbuf
            if nxt < n_chunks:
                wdesc[b].wait()
                gdesc[b] = pltpu.async_copy(
                    x_hbm.at[idx_v.at[nxt]], bufs[b], gsems[b])
        for c in range(max(0, n_chunks - nbuf), n_chunks):
            b = c % nbuf
            if wdesc[b] is not None:
                wdesc[b].wait()
                wdesc[b] = None

    return gk(xb, row_token_3d)


def _sc_combine_call(ys, pos0_3d, pos1_3d, n_tok, hidden):
    tok_per_w = n_tok // NW
    chunk = 16
    n_chunks = tok_per_w // chunk
    idx_rows = tok_per_w // LANES

    mesh = plsc.VectorSubcoreMesh(
        core_axis_name="c", subcore_axis_name="s",
        num_cores=SC_CORES, num_subcores=SC_SUBCORES)

    @functools.partial(
        pl.kernel, mesh=mesh,
        out_type=jax.ShapeDtypeStruct((n_tok, hidden), jnp.float32),
        scratch_types=[
            pltpu.VMEM((idx_rows, LANES), jnp.int32),
            pltpu.VMEM((idx_rows, LANES), jnp.int32),
            pltpu.VMEM((chunk, hidden), jnp.float32),
            pltpu.VMEM((chunk, hidden), jnp.float32),
            pltpu.SemaphoreType.DMA,
            pltpu.SemaphoreType.DMA,
        ],
    )
    def ck(ys_hbm, p0_hbm, p1_hbm, out_hbm, p0_v, p1_v, buf0, buf1,
           sem0, sem1):
        wid = lax.axis_index("s") * SC_CORES + lax.axis_index("c")
        base_tok = wid * tok_per_w
        pltpu.sync_copy(p0_hbm.at[wid], p0_v)
        pltpu.sync_copy(p1_hbm.at[wid], p1_v)
        n_vec = hidden // LANES
        for c in range(n_chunks):
            d0 = pltpu.async_copy(ys_hbm.at[p0_v.at[c]], buf0, sem0)
            d1 = pltpu.async_copy(ys_hbm.at[p1_v.at[c]], buf1, sem1)
            d0.wait()
            d1.wait()

            def add_row(r, _):
                def add_vec(v, _):
                    sl = pl.ds(v * LANES, LANES)
                    buf0[r, sl] = buf0[r, sl] + buf1[r, sl]
                    return 0
                lax.fori_loop(0, n_vec, add_vec, 0)
                return 0
            lax.fori_loop(0, chunk, add_row, 0)
            pltpu.sync_copy(buf0,
                            out_hbm.at[pl.ds(base_tok + c * chunk, chunk)])

    return ck(ys, pos0_3d, pos1_3d)


def _ffn_kernel(tile_expert_ref, tile_flag_ref, tile_imap_ref,
                xs_ref, w1g_ref, w1u_ref, w2_ref, w_ref,
                ys_ref, acc_ref, *, n_j):
    j = pl.program_id(1)

    @pl.when(tile_flag_ref[pl.program_id(0)] != 0)
    def _active():
        xs = xs_ref[...]
        w1g = w1g_ref[0, 0].astype(jnp.bfloat16)
        w1u = w1u_ref[0, 0].astype(jnp.bfloat16)
        gate = lax.dot_general(
            xs, w1g, (((1,), (1,)), ((), ())),
            preferred_element_type=jnp.float32)
        up = lax.dot_general(
            xs, w1u, (((1,), (1,)), ((), ())),
            preferred_element_type=jnp.float32)
        act = (gate * jax.nn.sigmoid(gate) * up).astype(jnp.bfloat16)
        w2b = w2_ref[0].astype(jnp.bfloat16)
        yj = lax.dot_general(
            act, w2b, (((1,), (1,)), ((), ())),
            preferred_element_type=jnp.float32)

        @pl.when(j == 0)
        def _init():
            acc_ref[...] = yj

        @pl.when(j > 0)
        def _acc():
            acc_ref[...] += yj

        @pl.when(j == n_j - 1)
        def _weight():
            ys_ref[...] = acc_ref[...] * w_ref[...]


def kernel(x, router_logits, w1, w2):
    n_tok, hidden = x.shape
    n_exp = w1.shape[0]
    d_ff = w2.shape[2]

    probs = jax.nn.softmax(router_logits.astype(jnp.float32), axis=-1)
    topk_w, topk_idx = lax.top_k(probs, TOP_K)
    topk_w = topk_w / jnp.sum(topk_w, axis=-1, keepdims=True)

    n_asn = n_tok * TOP_K
    e_flat = topk_idx.reshape(-1).astype(jnp.int32)
    w_flat = topk_w.reshape(-1)
    t_flat = jnp.repeat(jnp.arange(n_tok, dtype=jnp.int32), TOP_K)

    order = jnp.argsort(e_flat)
    e_s = e_flat[order]
    t_s = t_flat[order]
    w_s = w_flat[order]

    counts = jnp.bincount(e_flat, length=n_exp)
    padded = ((counts + TM - 1) // TM) * TM
    pstart = jnp.cumsum(padded) - padded
    gstart = jnp.cumsum(counts) - counts
    rank = jnp.arange(n_asn, dtype=jnp.int32) - gstart[e_s].astype(jnp.int32)
    dest = pstart[e_s].astype(jnp.int32) + rank

    n_tiles = n_asn // TM + n_exp
    n_rows_tc = n_tiles * TM
    gq = NW * GATHER_CHUNK
    n_rows = ((n_rows_tc + gq - 1) // gq) * gq
    row_token = jnp.zeros((n_rows,), jnp.int32).at[dest].set(t_s)
    row_weight = jnp.zeros((n_rows_tc, 1), jnp.float32).at[dest, 0].set(w_s)

    flat_pos = jnp.zeros((n_asn,), jnp.int32).at[order].set(dest)
    pos = flat_pos.reshape(n_tok, TOP_K)
    pos0 = pos[:, 0].reshape(NW, n_tok // NW // LANES, LANES)
    pos1 = pos[:, 1].reshape(NW, n_tok // NW // LANES, LANES)

    tile_start = jnp.arange(n_tiles, dtype=jnp.int32) * TM
    total_padded = jnp.sum(padded).astype(jnp.int32)
    tile_flag = (tile_start < total_padded).astype(jnp.int32)
    n_valid = jnp.maximum(total_padded // TM, 1).astype(jnp.int32)
    pend = (pstart + padded).astype(jnp.int32)
    raw_expert = jnp.clip(
        jnp.searchsorted(pend, tile_start, side='right'), 0, n_exp - 1
    ).astype(jnp.int32)
    e_last = e_s[-1].astype(jnp.int32)
    tile_expert = jnp.where(tile_flag == 1, raw_expert, e_last)
    tile_imap = jnp.where(tile_flag == 1,
                          jnp.arange(n_tiles, dtype=jnp.int32), n_valid - 1)

    xb = x.astype(jnp.bfloat16)
    xbi = lax.bitcast_convert_type(
        xb.reshape(n_tok, hidden // 2, 2), jnp.int32)
    xs_i = _sc_gather_call(
        xbi, row_token.reshape(NW, -1, GATHER_CHUNK),
        n_rows, hidden // 2)
    xs = lax.bitcast_convert_type(xs_i, jnp.bfloat16).reshape(n_rows, hidden)

    w1r = w1.reshape(n_exp, 2, d_ff, hidden)
    n_j = d_ff // DF_BLK

    def wmap(sel):
        def f(i, j, te, tf, ti):
            je = jnp.where(tf[i] == 0, n_j - 1, j)
            return (te[i], sel, je, 0)
        return f

    def w2map(i, j, te, tf, ti):
        je = jnp.where(tf[i] == 0, n_j - 1, j)
        return (te[i], 0, je)

    grid_spec = pltpu.PrefetchScalarGridSpec(
        num_scalar_prefetch=3,
        grid=(n_tiles, n_j),
        in_specs=[
            pl.BlockSpec((TM, hidden), lambda i, j, te, tf, ti: (ti[i], 0)),
            pl.BlockSpec((1, 1, DF_BLK, hidden), wmap(0)),
            pl.BlockSpec((1, 1, DF_BLK, hidden), wmap(1)),
            pl.BlockSpec((1, hidden, DF_BLK), w2map),
            pl.BlockSpec((TM, 1), lambda i, j, te, tf, ti: (ti[i], 0)),
        ],
        out_specs=pl.BlockSpec((TM, hidden),
                               lambda i, j, te, tf, ti: (ti[i], 0)),
        scratch_shapes=[pltpu.VMEM((TM, hidden), jnp.float32)],
    )

    ys = pl.pallas_call(
        functools.partial(_ffn_kernel, n_j=n_j),
        grid_spec=grid_spec,
        out_shape=jax.ShapeDtypeStruct((n_rows_tc, hidden), jnp.float32),
        compiler_params=pltpu.CompilerParams(
            dimension_semantics=("arbitrary", "arbitrary")),
    )(tile_expert, tile_flag, tile_imap,
      xs, w1r, w1r, w2, row_weight)

    out = _sc_combine_call(ys, pos0, pos1, n_tok, hidden)
    return out


# speedup vs baseline: 1.6120x; 1.6016x over previous
import functools

import jax
import jax.numpy as jnp
from jax import lax
from jax.experimental import pallas as pl
from jax.experimental.pallas import tpu as pltpu
from jax.experimental.pallas import tpu_sc as plsc

TOP_K = 2
TM = 256
N_JO = 2

SC_CORES = 2
SC_SUBCORES = 16
NW = SC_CORES * SC_SUBCORES
LANES = 16

GATHER_CHUNK = 48
GATHER_NBUF = 2


def _sc_gather_call(xb3, row_token_3d, n_rows):
    rows_per_w = n_rows // NW
    chunk = GATHER_CHUNK
    n_chunks = rows_per_w // chunk
    nbuf = GATHER_NBUF

    mesh = plsc.VectorSubcoreMesh(
        core_axis_name="c", subcore_axis_name="s",
        num_cores=SC_CORES, num_subcores=SC_SUBCORES)

    @functools.partial(
        pl.kernel, mesh=mesh,
        out_type=jax.ShapeDtypeStruct((n_rows, 8, 128), jnp.int32),
        scratch_types=(
            [pltpu.VMEM((n_chunks, chunk), jnp.int32)]
            + [pltpu.VMEM((chunk, 8, 128), jnp.int32) for _ in range(nbuf)]
            + [pltpu.SemaphoreType.DMA for _ in range(2 * nbuf)]
        ),
    )
    def gk(x_hbm, idx_hbm, out_hbm, idx_v, *bufs_sems):
        bufs = bufs_sems[:nbuf]
        gsems = bufs_sems[nbuf:2 * nbuf]
        wsems = bufs_sems[2 * nbuf:]
        wid = lax.axis_index("s") * SC_CORES + lax.axis_index("c")
        base_row = wid * rows_per_w
        pltpu.sync_copy(idx_hbm.at[wid], idx_v)

        gdesc = [None] * nbuf
        wdesc = [None] * nbuf
        for c in range(min(nbuf, n_chunks)):
            gdesc[c] = pltpu.async_copy(
                x_hbm.at[idx_v.at[c]], bufs[c], gsems[c])
        for c in range(n_chunks):
            b = c % nbuf
            gdesc[b].wait()
            wdesc[b] = pltpu.async_copy(
                bufs[b], out_hbm.at[pl.ds(base_row + c * chunk, chunk)],
                wsems[b])
            nxt = c + nbuf
            if nxt < n_chunks:
                wdesc[b].wait()
                gdesc[b] = pltpu.async_copy(
                    x_hbm.at[idx_v.at[nxt]], bufs[b], gsems[b])
        for c in range(max(0, n_chunks - nbuf), n_chunks):
            b = c % nbuf
            if wdesc[b] is not None:
                wdesc[b].wait()
                wdesc[b] = None

    return gk(xb3, row_token_3d)


def _sc_combine_call(ys, pos0_3d, pos1_3d, n_tok, hidden):
    tok_per_w = n_tok // NW
    chunk = 16
    n_chunks = tok_per_w // chunk
    idx_rows = tok_per_w // LANES

    mesh = plsc.VectorSubcoreMesh(
        core_axis_name="c", subcore_axis_name="s",
        num_cores=SC_CORES, num_subcores=SC_SUBCORES)

    @functools.partial(
        pl.kernel, mesh=mesh,
        out_type=jax.ShapeDtypeStruct((n_tok, hidden), jnp.float32),
        scratch_types=[
            pltpu.VMEM((idx_rows, LANES), jnp.int32),
            pltpu.VMEM((idx_rows, LANES), jnp.int32),
            pltpu.VMEM((chunk, hidden), jnp.float32),
            pltpu.VMEM((chunk, hidden), jnp.float32),
            pltpu.SemaphoreType.DMA,
            pltpu.SemaphoreType.DMA,
        ],
    )
    def ck(ys_hbm, p0_hbm, p1_hbm, out_hbm, p0_v, p1_v, buf0, buf1,
           sem0, sem1):
        wid = lax.axis_index("s") * SC_CORES + lax.axis_index("c")
        base_tok = wid * tok_per_w
        pltpu.sync_copy(p0_hbm.at[wid], p0_v)
        pltpu.sync_copy(p1_hbm.at[wid], p1_v)
        n_vec = hidden // LANES
        for c in range(n_chunks):
            d0 = pltpu.async_copy(ys_hbm.at[p0_v.at[c]], buf0, sem0)
            d1 = pltpu.async_copy(ys_hbm.at[p1_v.at[c]], buf1, sem1)
            d0.wait()
            d1.wait()

            def add_row(r, _):
                def add_vec(v, _):
                    sl = pl.ds(v * LANES, LANES)
                    buf0[r, sl] = buf0[r, sl] + buf1[r, sl]
                    return 0
                lax.fori_loop(0, n_vec, add_vec, 0)
                return 0
            lax.fori_loop(0, chunk, add_row, 0)
            pltpu.sync_copy(buf0,
                            out_hbm.at[pl.ds(base_tok + c * chunk, chunk)])

    return ck(ys, pos0_3d, pos1_3d)


def _act_kernel(tile_expert_ref, tile_flag_ref, tile_imap_ref,
                xs_ref, w1g_ref, w1u_ref, act_ref):
    @pl.when(tile_flag_ref[pl.program_id(0)] != 0)
    def _active():
        xs = xs_ref[...].astype(jnp.float32)
        gate = lax.dot_general(
            xs, w1g_ref[0, 0], (((1,), (1,)), ((), ())),
            preferred_element_type=jnp.float32)
        up = lax.dot_general(
            xs, w1u_ref[0, 0], (((1,), (1,)), ((), ())),
            preferred_element_type=jnp.float32)
        act = gate * jax.nn.sigmoid(gate) * up
        act_ref[...] = act.astype(jnp.bfloat16)


def _down_kernel(tile_expert_ref, tile_flag_ref, tile_imap_ref,
                 act_ref, w2_ref, w_ref, ys_ref):
    @pl.when(tile_flag_ref[pl.program_id(0)] != 0)
    def _active():
        act = act_ref[...].astype(jnp.float32)
        yj = lax.dot_general(
            act, w2_ref[0], (((1,), (1,)), ((), ())),
            preferred_element_type=jnp.float32)
        ys_ref[...] = yj * w_ref[...]


def kernel(x, router_logits, w1, w2):
    n_tok, hidden = x.shape
    n_exp = w1.shape[0]
    d_ff = w2.shape[2]
    d_half = d_ff // N_JO

    probs = jax.nn.softmax(router_logits.astype(jnp.float32), axis=-1)
    topk_w, topk_idx = lax.top_k(probs, TOP_K)
    topk_w = topk_w / jnp.sum(topk_w, axis=-1, keepdims=True)

    n_asn = n_tok * TOP_K
    e_flat = topk_idx.reshape(-1).astype(jnp.int32)
    w_flat = topk_w.reshape(-1)
    t_flat = jnp.repeat(jnp.arange(n_tok, dtype=jnp.int32), TOP_K)

    order = jnp.argsort(e_flat)
    e_s = e_flat[order]
    t_s = t_flat[order]
    w_s = w_flat[order]

    counts = jnp.bincount(e_flat, length=n_exp)
    padded = ((counts + TM - 1) // TM) * TM
    pstart = jnp.cumsum(padded) - padded
    gstart = jnp.cumsum(counts) - counts
    rank = jnp.arange(n_asn, dtype=jnp.int32) - gstart[e_s].astype(jnp.int32)
    dest = pstart[e_s].astype(jnp.int32) + rank

    n_tiles = n_asn // TM + n_exp
    n_rows = n_tiles * TM
    row_token = jnp.zeros((n_rows,), jnp.int32).at[dest].set(t_s)
    row_weight = jnp.zeros((n_rows, 1), jnp.float32).at[dest, 0].set(w_s)

    flat_pos = jnp.zeros((n_asn,), jnp.int32).at[order].set(dest)
    pos = flat_pos.reshape(n_tok, TOP_K)
    pos0 = pos[:, 0].reshape(NW, n_tok // NW // LANES, LANES)
    pos1 = pos[:, 1].reshape(NW, n_tok // NW // LANES, LANES)

    tile_start = jnp.arange(n_tiles, dtype=jnp.int32) * TM
    total_padded = jnp.sum(padded).astype(jnp.int32)
    tile_flag = (tile_start < total_padded).astype(jnp.int32)
    n_valid = jnp.maximum(total_padded // TM, 1).astype(jnp.int32)
    pend = (pstart + padded).astype(jnp.int32)
    raw_expert = jnp.clip(
        jnp.searchsorted(pend, tile_start, side='right'), 0, n_exp - 1
    ).astype(jnp.int32)
    e_last = e_s[-1].astype(jnp.int32)
    tile_expert = jnp.where(tile_flag == 1, raw_expert, e_last)
    tile_imap = jnp.where(tile_flag == 1,
                          jnp.arange(n_tiles, dtype=jnp.int32), n_valid - 1)

    xb3 = lax.bitcast_convert_type(
        x.astype(jnp.bfloat16).reshape(n_tok, 8, 128, 2), jnp.int32)
    xs3 = _sc_gather_call(
        xb3, row_token.reshape(NW, -1, GATHER_CHUNK), n_rows)
    xs = lax.bitcast_convert_type(xs3, jnp.bfloat16).reshape(n_rows, hidden)

    w1r = w1.reshape(n_exp, 2, d_ff, hidden)

    def tile_map(i, te, tf, ti):
        return (ti[i], 0)

    act = pl.pallas_call(
        _act_kernel,
        grid_spec=pltpu.PrefetchScalarGridSpec(
            num_scalar_prefetch=3,
            grid=(n_tiles,),
            in_specs=[
                pl.BlockSpec((TM, hidden), tile_map),
                pl.BlockSpec((1, 1, d_ff, hidden),
                             lambda i, te, tf, ti: (te[i], 0, 0, 0)),
                pl.BlockSpec((1, 1, d_ff, hidden),
                             lambda i, te, tf, ti: (te[i], 1, 0, 0)),
            ],
            out_specs=pl.BlockSpec((TM, d_ff), tile_map),
        ),
        out_shape=jax.ShapeDtypeStruct((n_rows, d_ff), jnp.bfloat16),
        compiler_params=pltpu.CompilerParams(
            dimension_semantics=("arbitrary",)),
    )(tile_expert, tile_flag, tile_imap, xs, w1r, w1r)

    ys = pl.pallas_call(
        _down_kernel,
        grid_spec=pltpu.PrefetchScalarGridSpec(
            num_scalar_prefetch=3,
            grid=(n_tiles,),
            in_specs=[
                pl.BlockSpec((TM, d_ff), tile_map),
                pl.BlockSpec((1, hidden, d_ff),
                             lambda i, te, tf, ti: (te[i], 0, 0)),
                pl.BlockSpec((TM, 1), tile_map),
            ],
            out_specs=pl.BlockSpec((TM, hidden), tile_map),
        ),
        out_shape=jax.ShapeDtypeStruct((n_rows, hidden), jnp.float32),
        compiler_params=pltpu.CompilerParams(
            dimension_semantics=("arbitrary",)),
    )(tile_expert, tile_flag, tile_imap, act, w2, row_weight)

    out = _sc_combine_call(ys, pos0, pos1, n_tok, hidden)
    return out


# speedup vs baseline: 1.7125x; 1.0624x over previous
import functools

import jax
import jax.numpy as jnp
from jax import lax
from jax.experimental import pallas as pl
from jax.experimental.pallas import tpu as pltpu
from jax.experimental.pallas import tpu_sc as plsc

TOP_K = 2
TM = 256
N_JO = 2

SC_CORES = 2
SC_SUBCORES = 16
NW = SC_CORES * SC_SUBCORES
LANES = 16

GATHER_CHUNK = 48
GATHER_NBUF = 2


def _sc_gather_call(xb3, row_token_3d, n_rows):
    rows_per_w = n_rows // NW
    chunk = GATHER_CHUNK
    n_chunks = rows_per_w // chunk
    nbuf = GATHER_NBUF

    mesh = plsc.VectorSubcoreMesh(
        core_axis_name="c", subcore_axis_name="s",
        num_cores=SC_CORES, num_subcores=SC_SUBCORES)

    @functools.partial(
        pl.kernel, mesh=mesh,
        out_type=jax.ShapeDtypeStruct((n_rows, 8, 128), jnp.int32),
        scratch_types=(
            [pltpu.VMEM((n_chunks, chunk), jnp.int32)]
            + [pltpu.VMEM((chunk, 8, 128), jnp.int32) for _ in range(nbuf)]
            + [pltpu.SemaphoreType.DMA for _ in range(2 * nbuf)]
        ),
    )
    def gk(x_hbm, idx_hbm, out_hbm, idx_v, *bufs_sems):
        bufs = bufs_sems[:nbuf]
        gsems = bufs_sems[nbuf:2 * nbuf]
        wsems = bufs_sems[2 * nbuf:]
        wid = lax.axis_index("s") * SC_CORES + lax.axis_index("c")
        base_row = wid * rows_per_w
        pltpu.sync_copy(idx_hbm.at[wid], idx_v)

        gdesc = [None] * nbuf
        wdesc = [None] * nbuf
        for c in range(min(nbuf, n_chunks)):
            gdesc[c] = pltpu.async_copy(
                x_hbm.at[idx_v.at[c]], bufs[c], gsems[c])
        for c in range(n_chunks):
            b = c % nbuf
            gdesc[b].wait()
            wdesc[b] = pltpu.async_copy(
                bufs[b], out_hbm.at[pl.ds(base_row + c * chunk, chunk)],
                wsems[b])
            nxt = c + nbuf
            if nxt < n_chunks:
                wdesc[b].wait()
                gdesc[b] = pltpu.async_copy(
                    x_hbm.at[idx_v.at[nxt]], bufs[b], gsems[b])
        for c in range(max(0, n_chunks - nbuf), n_chunks):
            b = c % nbuf
            if wdesc[b] is not None:
                wdesc[b].wait()
                wdesc[b] = None

    return gk(xb3, row_token_3d)


def _sc_combine_call(ys, pos0_3d, pos1_3d, n_tok, hidden):
    tok_per_w = n_tok // NW
    chunk = 16
    n_chunks = tok_per_w // chunk
    idx_rows = tok_per_w // LANES

    mesh = plsc.VectorSubcoreMesh(
        core_axis_name="c", subcore_axis_name="s",
        num_cores=SC_CORES, num_subcores=SC_SUBCORES)

    @functools.partial(
        pl.kernel, mesh=mesh,
        out_type=jax.ShapeDtypeStruct((n_tok, hidden), jnp.float32),
        scratch_types=[
            pltpu.VMEM((idx_rows, LANES), jnp.int32),
            pltpu.VMEM((idx_rows, LANES), jnp.int32),
            pltpu.VMEM((chunk, hidden), jnp.float32),
            pltpu.VMEM((chunk, hidden), jnp.float32),
            pltpu.SemaphoreType.DMA,
            pltpu.SemaphoreType.DMA,
        ],
    )
    def ck(ys_hbm, p0_hbm, p1_hbm, out_hbm, p0_v, p1_v, buf0, buf1,
           sem0, sem1):
        wid = lax.axis_index("s") * SC_CORES + lax.axis_index("c")
        base_tok = wid * tok_per_w
        pltpu.sync_copy(p0_hbm.at[wid], p0_v)
        pltpu.sync_copy(p1_hbm.at[wid], p1_v)
        n_vec = hidden // LANES
        for c in range(n_chunks):
            d0 = pltpu.async_copy(ys_hbm.at[p0_v.at[c]], buf0, sem0)
            d1 = pltpu.async_copy(ys_hbm.at[p1_v.at[c]], buf1, sem1)
            d0.wait()
            d1.wait()

            def add_row(r, _):
                def add_vec(v, _):
                    sl = pl.ds(v * LANES, LANES)
                    buf0[r, sl] = buf0[r, sl] + buf1[r, sl]
                    return 0
                lax.fori_loop(0, n_vec, add_vec, 0)
                return 0
            lax.fori_loop(0, chunk, add_row, 0)
            pltpu.sync_copy(buf0,
                            out_hbm.at[pl.ds(base_tok + c * chunk, chunk)])

    return ck(ys, pos0_3d, pos1_3d)


def _act_kernel(tile_expert_ref, tile_flag_ref, tile_imap_ref,
                xs_ref, w1g_ref, w1u_ref, act_ref):
    @pl.when(tile_flag_ref[pl.program_id(0)] != 0)
    def _active():
        xs = xs_ref[...].astype(jnp.float32)
        gate = lax.dot_general(
            xs, w1g_ref[0, 0], (((1,), (1,)), ((), ())),
            preferred_element_type=jnp.float32)
        up = lax.dot_general(
            xs, w1u_ref[0, 0], (((1,), (1,)), ((), ())),
            preferred_element_type=jnp.float32)
        act = gate * jax.nn.sigmoid(gate) * up
        act_ref[...] = act.astype(jnp.bfloat16)


def _down_kernel(tile_expert_ref, tile_flag_ref, tile_imap_ref,
                 act_ref, w2_ref, w_ref, ys_ref):
    @pl.when(tile_flag_ref[pl.program_id(0)] != 0)
    def _active():
        act = act_ref[...].astype(jnp.float32)
        yj = lax.dot_general(
            act, w2_ref[0], (((1,), (1,)), ((), ())),
            preferred_element_type=jnp.float32)
        ys_ref[...] = yj * w_ref[...]


def kernel(x, router_logits, w1, w2):
    n_tok, hidden = x.shape
    n_exp = w1.shape[0]
    d_ff = w2.shape[2]
    d_half = d_ff // N_JO

    probs = jax.nn.softmax(router_logits.astype(jnp.float32), axis=-1)
    topk_w, topk_idx = lax.top_k(probs, TOP_K)
    topk_w = topk_w / jnp.sum(topk_w, axis=-1, keepdims=True)

    n_asn = n_tok * TOP_K
    e_flat = topk_idx.reshape(-1).astype(jnp.int32)
    w_flat = topk_w.reshape(-1)
    t_flat = jnp.repeat(jnp.arange(n_tok, dtype=jnp.int32), TOP_K)

    oh = (e_flat[:, None] == jnp.arange(n_exp, dtype=jnp.int32)[None, :])
    ohi = oh.astype(jnp.int32)
    counts = jnp.sum(ohi, axis=0)
    rank = jnp.sum(jnp.cumsum(ohi, axis=0) * ohi, axis=1) - 1
    padded = ((counts + TM - 1) // TM) * TM
    pstart = jnp.cumsum(padded) - padded
    dest = jnp.sum(ohi * pstart[None, :], axis=1).astype(jnp.int32) + rank

    n_tiles = n_asn // TM + n_exp
    n_rows = n_tiles * TM
    row_token = jnp.zeros((n_rows,), jnp.int32).at[dest].set(t_flat)
    row_weight = jnp.zeros((n_rows, 1), jnp.float32).at[dest, 0].set(w_flat)

    pos = dest.reshape(n_tok, TOP_K)
    pos0 = pos[:, 0].reshape(NW, n_tok // NW // LANES, LANES)
    pos1 = pos[:, 1].reshape(NW, n_tok // NW // LANES, LANES)

    tile_start = jnp.arange(n_tiles, dtype=jnp.int32) * TM
    total_padded = jnp.sum(padded).astype(jnp.int32)
    tile_flag = (tile_start < total_padded).astype(jnp.int32)
    n_valid = jnp.maximum(total_padded // TM, 1).astype(jnp.int32)
    pend = (pstart + padded).astype(jnp.int32)
    raw_expert = jnp.clip(
        jnp.sum((pend[None, :] <= tile_start[:, None]).astype(jnp.int32),
                axis=1),
        0, n_exp - 1).astype(jnp.int32)
    eids = jnp.arange(n_exp, dtype=jnp.int32)
    e_last = jnp.max(jnp.where(counts > 0, eids, -1)).astype(jnp.int32)
    tile_expert = jnp.where(tile_flag == 1, raw_expert, e_last)
    tile_imap = jnp.where(tile_flag == 1,
                          jnp.arange(n_tiles, dtype=jnp.int32), n_valid - 1)

    xb3 = lax.bitcast_convert_type(
        x.astype(jnp.bfloat16).reshape(n_tok, 8, 128, 2), jnp.int32)
    xs3 = _sc_gather_call(
        xb3, row_token.reshape(NW, -1, GATHER_CHUNK), n_rows)
    xs = lax.bitcast_convert_type(xs3, jnp.bfloat16).reshape(n_rows, hidden)

    w1r = w1.reshape(n_exp, 2, d_ff, hidden)

    def tile_map(i, te, tf, ti):
        return (ti[i], 0)

    act = pl.pallas_call(
        _act_kernel,
        grid_spec=pltpu.PrefetchScalarGridSpec(
            num_scalar_prefetch=3,
            grid=(n_tiles,),
            in_specs=[
                pl.BlockSpec((TM, hidden), tile_map),
                pl.BlockSpec((1, 1, d_ff, hidden),
                             lambda i, te, tf, ti: (te[i], 0, 0, 0)),
                pl.BlockSpec((1, 1, d_ff, hidden),
                             lambda i, te, tf, ti: (te[i], 1, 0, 0)),
            ],
            out_specs=pl.BlockSpec((TM, d_ff), tile_map),
        ),
        out_shape=jax.ShapeDtypeStruct((n_rows, d_ff), jnp.bfloat16),
        compiler_params=pltpu.CompilerParams(
            dimension_semantics=("arbitrary",)),
    )(tile_expert, tile_flag, tile_imap, xs, w1r, w1r)

    ys = pl.pallas_call(
        _down_kernel,
        grid_spec=pltpu.PrefetchScalarGridSpec(
            num_scalar_prefetch=3,
            grid=(n_tiles,),
            in_specs=[
                pl.BlockSpec((TM, d_ff), tile_map),
                pl.BlockSpec((1, hidden, d_ff),
                             lambda i, te, tf, ti: (te[i], 0, 0)),
                pl.BlockSpec((TM, 1), tile_map),
            ],
            out_specs=pl.BlockSpec((TM, hidden), tile_map),
        ),
        out_shape=jax.ShapeDtypeStruct((n_rows, hidden), jnp.float32),
        compiler_params=pltpu.CompilerParams(
            dimension_semantics=("arbitrary",)),
    )(tile_expert, tile_flag, tile_imap, act, w2, row_weight)

    out = _sc_combine_call(ys, pos0, pos1, n_tok, hidden)
    return out


# speedup vs baseline: 2.8757x; 1.6792x over previous
import functools

import jax
import jax.numpy as jnp
from jax import lax
from jax.experimental import pallas as pl
from jax.experimental.pallas import tpu as pltpu
from jax.experimental.pallas import tpu_sc as plsc

TOP_K = 2
TM = 256
N_JO = 2

SC_CORES = 2
SC_SUBCORES = 16
NW = SC_CORES * SC_SUBCORES
LANES = 16

GATHER_CHUNK = 24
GATHER_NBUF = 2


def _sc_gather_call(xr3, row_token_3d, n_rows):
    rows_per_w = n_rows // NW
    chunk = GATHER_CHUNK
    n_chunks = rows_per_w // chunk
    nbuf = GATHER_NBUF

    mesh = plsc.VectorSubcoreMesh(
        core_axis_name="c", subcore_axis_name="s",
        num_cores=SC_CORES, num_subcores=SC_SUBCORES)

    @functools.partial(
        pl.kernel, mesh=mesh,
        out_type=jax.ShapeDtypeStruct((n_rows, 16, 128), jnp.float32),
        scratch_types=(
            [pltpu.VMEM((n_chunks, chunk), jnp.int32)]
            + [pltpu.VMEM((chunk, 16, 128), jnp.float32) for _ in range(nbuf)]
            + [pltpu.SemaphoreType.DMA for _ in range(2 * nbuf)]
        ),
    )
    def gk(x_hbm, idx_hbm, out_hbm, idx_v, *bufs_sems):
        bufs = bufs_sems[:nbuf]
        gsems = bufs_sems[nbuf:2 * nbuf]
        wsems = bufs_sems[2 * nbuf:]
        wid = lax.axis_index("s") * SC_CORES + lax.axis_index("c")
        base_row = wid * rows_per_w
        pltpu.sync_copy(idx_hbm.at[wid], idx_v)

        gdesc = [None] * nbuf
        wdesc = [None] * nbuf
        for c in range(min(nbuf, n_chunks)):
            gdesc[c] = pltpu.async_copy(
                x_hbm.at[idx_v.at[c]], bufs[c], gsems[c])
        for c in range(n_chunks):
            b = c % nbuf
            gdesc[b].wait()
            wdesc[b] = pltpu.async_copy(
                bufs[b], out_hbm.at[pl.ds(base_row + c * chunk, chunk)],
                wsems[b])
            nxt = c + nbuf
            if nxt < n_chunks:
                wdesc[b].wait()
                gdesc[b] = pltpu.async_copy(
                    x_hbm.at[idx_v.at[nxt]], bufs[b], gsems[b])
        for c in range(max(0, n_chunks - nbuf), n_chunks):
            b = c % nbuf
            if wdesc[b] is not None:
                wdesc[b].wait()
                wdesc[b] = None

    return gk(xr3, row_token_3d)


def _sc_combine_call(ys, pos0_3d, pos1_3d, n_tok, hidden):
    tok_per_w = n_tok // NW
    chunk = 16
    n_chunks = tok_per_w // chunk
    idx_rows = tok_per_w // LANES

    mesh = plsc.VectorSubcoreMesh(
        core_axis_name="c", subcore_axis_name="s",
        num_cores=SC_CORES, num_subcores=SC_SUBCORES)

    @functools.partial(
        pl.kernel, mesh=mesh,
        out_type=jax.ShapeDtypeStruct((n_tok, hidden), jnp.float32),
        scratch_types=[
            pltpu.VMEM((idx_rows, LANES), jnp.int32),
            pltpu.VMEM((idx_rows, LANES), jnp.int32),
            pltpu.VMEM((chunk, hidden), jnp.float32),
            pltpu.VMEM((chunk, hidden), jnp.float32),
            pltpu.SemaphoreType.DMA,
            pltpu.SemaphoreType.DMA,
        ],
    )
    def ck(ys_hbm, p0_hbm, p1_hbm, out_hbm, p0_v, p1_v, buf0, buf1,
           sem0, sem1):
        wid = lax.axis_index("s") * SC_CORES + lax.axis_index("c")
        base_tok = wid * tok_per_w
        pltpu.sync_copy(p0_hbm.at[wid], p0_v)
        pltpu.sync_copy(p1_hbm.at[wid], p1_v)
        n_vec = hidden // LANES
        for c in range(n_chunks):
            d0 = pltpu.async_copy(ys_hbm.at[p0_v.at[c]], buf0, sem0)
            d1 = pltpu.async_copy(ys_hbm.at[p1_v.at[c]], buf1, sem1)
            d0.wait()
            d1.wait()

            def add_row(r, _):
                def add_vec(v, _):
                    sl = pl.ds(v * LANES, LANES)
                    buf0[r, sl] = buf0[r, sl] + buf1[r, sl]
                    return 0
                lax.fori_loop(0, n_vec, add_vec, 0)
                return 0
            lax.fori_loop(0, chunk, add_row, 0)
            pltpu.sync_copy(buf0,
                            out_hbm.at[pl.ds(base_tok + c * chunk, chunk)])

    return ck(ys, pos0_3d, pos1_3d)


def _act_kernel(tile_expert_ref, tile_flag_ref, tile_imap_ref,
                xs_ref, w1g_ref, w1u_ref, act_ref):
    @pl.when(tile_flag_ref[pl.program_id(0)] != 0)
    def _active():
        xs = xs_ref[...].astype(jnp.float32)
        gate = lax.dot_general(
            xs, w1g_ref[0, 0], (((1,), (1,)), ((), ())),
            preferred_element_type=jnp.float32)
        up = lax.dot_general(
            xs, w1u_ref[0, 0], (((1,), (1,)), ((), ())),
            preferred_element_type=jnp.float32)
        act = gate * jax.nn.sigmoid(gate) * up
        act_ref[...] = act.astype(jnp.bfloat16)


def _down_kernel(tile_expert_ref, tile_flag_ref, tile_imap_ref,
                 row_token_ref, act_ref, w2_ref, w_ref, out_ref, ys_ref):
    i = pl.program_id(0)

    @pl.when(i == 0)
    def _zero():
        out_ref[...] = jnp.zeros_like(out_ref)

    @pl.when(tile_flag_ref[i] != 0)
    def _active():
        act = act_ref[...].astype(jnp.float32)
        yj = lax.dot_general(
            act, w2_ref[0], (((1,), (1,)), ((), ())),
            preferred_element_type=jnp.float32)
        ys_ref[...] = yj * w_ref[...]

        def body(r, _):
            tok = row_token_ref[i * TM + r]
            out_ref[pl.ds(tok, 1), :] += ys_ref[pl.ds(r, 1), :]
            return 0
        lax.fori_loop(0, TM, body, 0, unroll=8)


def kernel(x, router_logits, w1, w2):
    n_tok, hidden = x.shape
    n_exp = w1.shape[0]
    d_ff = w2.shape[2]
    d_half = d_ff // N_JO

    probs = jax.nn.softmax(router_logits.astype(jnp.float32), axis=-1)
    topk_w, topk_idx = lax.top_k(probs, TOP_K)
    topk_w = topk_w / jnp.sum(topk_w, axis=-1, keepdims=True)

    n_asn = n_tok * TOP_K
    e_flat = topk_idx.reshape(-1).astype(jnp.int32)
    w_flat = topk_w.reshape(-1)
    t_flat = jnp.repeat(jnp.arange(n_tok, dtype=jnp.int32), TOP_K)

    oh = (e_flat[:, None] == jnp.arange(n_exp, dtype=jnp.int32)[None, :])
    ohi = oh.astype(jnp.int32)
    counts = jnp.sum(ohi, axis=0)
    rank = jnp.sum(jnp.cumsum(ohi, axis=0) * ohi, axis=1) - 1
    padded = ((counts + TM - 1) // TM) * TM
    pstart = jnp.cumsum(padded) - padded
    dest = jnp.sum(ohi * pstart[None, :], axis=1).astype(jnp.int32) + rank

    n_tiles = n_asn // TM + n_exp
    n_rows = n_tiles * TM
    row_token = jnp.zeros((n_rows,), jnp.int32).at[dest].set(t_flat)
    row_weight = jnp.zeros((n_rows, 1), jnp.float32).at[dest, 0].set(w_flat)

    tile_start = jnp.arange(n_tiles, dtype=jnp.int32) * TM
    total_padded = jnp.sum(padded).astype(jnp.int32)
    tile_flag = (tile_start < total_padded).astype(jnp.int32)
    n_valid = jnp.maximum(total_padded // TM, 1).astype(jnp.int32)
    pend = (pstart + padded).astype(jnp.int32)
    raw_expert = jnp.clip(
        jnp.sum((pend[None, :] <= tile_start[:, None]).astype(jnp.int32),
                axis=1),
        0, n_exp - 1).astype(jnp.int32)
    eids = jnp.arange(n_exp, dtype=jnp.int32)
    e_last = jnp.max(jnp.where(counts > 0, eids, -1)).astype(jnp.int32)
    tile_expert = jnp.where(tile_flag == 1, raw_expert, e_last)
    tile_imap = jnp.where(tile_flag == 1,
                          jnp.arange(n_tiles, dtype=jnp.int32), n_valid - 1)

    xr3 = x.reshape(n_tok, 16, 128)
    xs3 = _sc_gather_call(
        xr3, row_token.reshape(NW, -1, GATHER_CHUNK), n_rows)
    xs = xs3.reshape(n_rows, hidden)

    w1r = w1.reshape(n_exp, 2, d_ff, hidden)

    def tile_map(i, te, tf, ti):
        return (ti[i], 0)

    act = pl.pallas_call(
        _act_kernel,
        grid_spec=pltpu.PrefetchScalarGridSpec(
            num_scalar_prefetch=3,
            grid=(n_tiles,),
            in_specs=[
                pl.BlockSpec((TM, hidden), tile_map),
                pl.BlockSpec((1, 1, d_ff, hidden),
                             lambda i, te, tf, ti: (te[i], 0, 0, 0)),
                pl.BlockSpec((1, 1, d_ff, hidden),
                             lambda i, te, tf, ti: (te[i], 1, 0, 0)),
            ],
            out_specs=pl.BlockSpec((TM, d_ff), tile_map),
        ),
        out_shape=jax.ShapeDtypeStruct((n_rows, d_ff), jnp.bfloat16),
        compiler_params=pltpu.CompilerParams(
            dimension_semantics=("arbitrary",)),
    )(tile_expert, tile_flag, tile_imap, xs, w1r, w1r)

    out = pl.pallas_call(
        _down_kernel,
        grid_spec=pltpu.PrefetchScalarGridSpec(
            num_scalar_prefetch=4,
            grid=(n_tiles,),
            in_specs=[
                pl.BlockSpec((TM, d_ff),
                             lambda i, te, tf, ti, rt: (ti[i], 0)),
                pl.BlockSpec((1, hidden, d_ff),
                             lambda i, te, tf, ti, rt: (te[i], 0, 0)),
                pl.BlockSpec((TM, 1),
                             lambda i, te, tf, ti, rt: (ti[i], 0)),
            ],
            out_specs=pl.BlockSpec((n_tok, hidden),
                                   lambda i, te, tf, ti, rt: (0, 0)),
            scratch_shapes=[pltpu.VMEM((TM, hidden), jnp.float32)],
        ),
        out_shape=jax.ShapeDtypeStruct((n_tok, hidden), jnp.float32),
        compiler_params=pltpu.CompilerParams(
            dimension_semantics=("arbitrary",)),
    )(tile_expert, tile_flag, tile_imap, row_token, act, w2, row_weight)
    return out
